# baseline (speedup 1.0000x reference)
import jax
import jax.numpy as jnp
from jax import lax
from jax.experimental import pallas as pl
from jax.experimental.pallas import tpu as pltpu
from jax.experimental.pallas import tpu_sc as plsc

NC = 2
NS = 16
NW = NC * NS

B_ROWS, T_STEPS = 4096, 200
D = 64
ROWS_PER_W = B_ROWS // NW
NBUF = 4
N_GROUPS = ROWS_PER_W // NBUF


def _gather_body(idx_hbm, table_hbm, out_hbm, idx_v, rows_v, gsems, osems):
    wid = lax.axis_index("s") * NC + lax.axis_index("c")
    base_row = wid * ROWS_PER_W
    pltpu.sync_copy(idx_hbm.at[pl.ds(base_row, ROWS_PER_W)], idx_v)

    def start_gather(j, b):
        pltpu.async_copy(table_hbm.at[idx_v.at[j]], rows_v.at[b], gsems.at[b])

    def wait_gather(b):
        pltpu.make_async_copy(
            table_hbm.at[idx_v.at[0]], rows_v.at[b], gsems.at[b]
        ).wait()

    def start_out(j, b):
        pltpu.async_copy(rows_v.at[b], out_hbm.at[base_row + j], osems.at[b])

    def wait_out(b):
        pltpu.make_async_copy(
            rows_v.at[b], out_hbm.at[0], osems.at[b]
        ).wait()

    for b in range(NBUF):
        start_gather(b, b)

    @pl.loop(0, N_GROUPS - 1)
    def _(g):
        base = g * NBUF
        for b in range(NBUF):
            wait_gather(b)
            start_out(base + b, b)
        for b in range(NBUF):
            wait_out(b)
            start_gather(base + NBUF + b, b)

    base = (N_GROUPS - 1) * NBUF
    for b in range(NBUF):
        wait_gather(b)
        start_out(base + b, b)
    for b in range(NBUF):
        wait_out(b)


@jax.jit
def kernel(time_steps, W):
    mesh = plsc.VectorSubcoreMesh(core_axis_name="c", subcore_axis_name="s")
    out = pl.kernel(
        _gather_body,
        out_type=jax.ShapeDtypeStruct((B_ROWS, T_STEPS, D), jnp.float32),
        mesh=mesh,
        scratch_types=[
            pltpu.VMEM((ROWS_PER_W, T_STEPS), jnp.int32),
            pltpu.VMEM((NBUF, T_STEPS, D), jnp.float32),
            pltpu.SemaphoreType.DMA((NBUF,)),
            pltpu.SemaphoreType.DMA((NBUF,)),
        ],
        compiler_params=pltpu.CompilerParams(use_tc_tiling_on_sc=False),
    )(time_steps.astype(jnp.int32), W)
    flat = lax.optimization_barrier(out.reshape(B_ROWS * T_STEPS * D // 128, 128))
    return flat.reshape(B_ROWS, T_STEPS, D)


# speedup vs baseline: 1.2891x; 1.2891x over previous
import jax
import jax.numpy as jnp
from jax import lax
from jax.experimental import pallas as pl
from jax.experimental.pallas import tpu as pltpu
from jax.experimental.pallas import tpu_sc as plsc

NC = 2
NS = 16
NW = NC * NS

B_ROWS, T_STEPS = 4096, 200
D = 64
DP = 128
ROWS_PER_W = B_ROWS // NW
IDX_PER_W = ROWS_PER_W * T_STEPS
NBUF = 2
N_GROUPS = ROWS_PER_W // NBUF


def _gather_body(idx_hbm, table_hbm, out_hbm, idx_v, rows_v, gsems, osems):
    wid = lax.axis_index("s") * NC + lax.axis_index("c")
    base_row = wid * ROWS_PER_W
    pltpu.sync_copy(idx_hbm.at[pl.ds(wid * IDX_PER_W, IDX_PER_W)], idx_v)

    def start_gather(j, b):
        pltpu.async_copy(
            table_hbm.at[idx_v.at[pl.ds(j * T_STEPS, T_STEPS)]],
            rows_v.at[b],
            gsems.at[b],
        )

    def wait_gather(b):
        pltpu.make_async_copy(
            table_hbm.at[idx_v.at[pl.ds(0, T_STEPS)]], rows_v.at[b], gsems.at[b]
        ).wait()

    def start_out(j, b):
        pltpu.async_copy(rows_v.at[b], out_hbm.at[base_row + j], osems.at[b])

    def wait_out(b):
        pltpu.make_async_copy(
            rows_v.at[b], out_hbm.at[0], osems.at[b]
        ).wait()

    for b in range(NBUF):
        start_gather(b, b)

    @pl.loop(0, N_GROUPS - 1)
    def _(g):
        base = g * NBUF
        for b in range(NBUF):
            wait_gather(b)
            start_out(base + b, b)
        for b in range(NBUF):
            wait_out(b)
            start_gather(base + NBUF + b, b)

    base = (N_GROUPS - 1) * NBUF
    for b in range(NBUF):
        wait_gather(b)
        start_out(base + b, b)
    for b in range(NBUF):
        wait_out(b)


@jax.jit
def kernel(time_steps, W):
    W_pad = jnp.pad(W, ((0, 0), (0, DP - D)))
    idx = time_steps.astype(jnp.int32).reshape(-1)
    mesh = plsc.VectorSubcoreMesh(core_axis_name="c", subcore_axis_name="s")
    out = pl.kernel(
        _gather_body,
        out_type=jax.ShapeDtypeStruct((B_ROWS, T_STEPS, DP), jnp.float32),
        mesh=mesh,
        scratch_types=[
            pltpu.VMEM((IDX_PER_W,), jnp.int32),
            pltpu.VMEM((NBUF, T_STEPS, DP), jnp.float32),
            pltpu.SemaphoreType.DMA((NBUF,)),
            pltpu.SemaphoreType.DMA((NBUF,)),
        ],
        compiler_params=pltpu.CompilerParams(use_tc_tiling_on_sc=True),
    )(idx, W_pad)
    return out[:, :, :D]


# speedup vs baseline: 1.3108x; 1.0168x over previous
import jax
import jax.numpy as jnp
from jax import lax
from jax.experimental import pallas as pl
from jax.experimental.pallas import tpu as pltpu
from jax.experimental.pallas import tpu_sc as plsc

NC = 2
NS = 16
NW = NC * NS

B_ROWS, T_STEPS = 4096, 200
D = 64
DP = 128
ROWS_PER_W = B_ROWS // NW
IDX_PER_W = ROWS_PER_W * T_STEPS
NBUF = 4
N_GROUPS = ROWS_PER_W // NBUF


def _gather_body(idx_hbm, table_hbm, out_hbm, idx_v, rows_v, gsems, osems):
    wid = lax.axis_index("s") * NC + lax.axis_index("c")
    base_row = wid * ROWS_PER_W
    pltpu.sync_copy(idx_hbm.at[pl.ds(wid * IDX_PER_W, IDX_PER_W)], idx_v)

    def start_gather(j, b):
        pltpu.async_copy(
            table_hbm.at[idx_v.at[pl.ds(j * T_STEPS, T_STEPS)]],
            rows_v.at[b],
            gsems.at[b],
        )

    def wait_gather(b):
        pltpu.make_async_copy(
            table_hbm.at[idx_v.at[pl.ds(0, T_STEPS)]], rows_v.at[b], gsems.at[b]
        ).wait()

    def start_out(j, b):
        pltpu.async_copy(rows_v.at[b], out_hbm.at[base_row + j], osems.at[b])

    def wait_out(b):
        pltpu.make_async_copy(
            rows_v.at[b], out_hbm.at[0], osems.at[b]
        ).wait()

    for b in range(NBUF):
        start_gather(b, b)

    @pl.loop(0, N_GROUPS - 1)
    def _(g):
        base = g * NBUF
        for b in range(NBUF):
            wait_gather(b)
            start_out(base + b, b)
        for b in range(NBUF):
            wait_out(b)
            start_gather(base + NBUF + b, b)

    base = (N_GROUPS - 1) * NBUF
    for b in range(NBUF):
        wait_gather(b)
        start_out(base + b, b)
    for b in range(NBUF):
        wait_out(b)


@jax.jit
def kernel(time_steps, W):
    W_pad = jnp.pad(W, ((0, 0), (0, DP - D)))
    idx = time_steps.astype(jnp.int32).reshape(-1)
    mesh = plsc.VectorSubcoreMesh(core_axis_name="c", subcore_axis_name="s")
    out = pl.kernel(
        _gather_body,
        out_type=jax.ShapeDtypeStruct((B_ROWS, T_STEPS, DP), jnp.float32),
        mesh=mesh,
        scratch_types=[
            pltpu.VMEM((IDX_PER_W,), jnp.int32),
            pltpu.VMEM((NBUF, T_STEPS, DP), jnp.float32),
            pltpu.SemaphoreType.DMA((NBUF,)),
            pltpu.SemaphoreType.DMA((NBUF,)),
        ],
        compiler_params=pltpu.CompilerParams(use_tc_tiling_on_sc=True),
    )(idx, W_pad)
    return out[:, :, :D]
